# baseline (speedup 1.0000x reference)
import functools

import jax
import jax.numpy as jnp
from jax import lax
from jax.experimental import pallas as pl
from jax.experimental.pallas import tpu as pltpu
from jax.experimental.pallas import tpu_sc as plsc

NC = 2
NS = 16
L = 16
NW = NC * NS

CHUNK = 100
SUPER = 4
NTRI = 3
NBUF = 3


def _sc_side_partials(n_nodes: int, n_edges: int, d: int):
    assert d % L == 0
    assert n_edges % (NW * CHUNK * SUPER) == 0
    e_per_w = n_edges // NW
    n_chunks = e_per_w // CHUNK
    n_super = n_chunks // SUPER
    assert n_chunks >= 4
    assert n_nodes % 16 == 0
    base_rows = (n_nodes // NS) // 16 * 16
    zrows = CHUNK // 8 * 8
    n_zdma = base_rows // zrows
    rem = base_rows - n_zdma * zrows
    last_rem = n_nodes - (NS - 1) * base_rows - n_zdma * zrows
    assert base_rows % 8 == 0 and rem % 8 == 0 and last_rem % 8 == 0
    assert 0 < rem <= zrows and 0 < last_rem <= zrows

    mesh = plsc.VectorSubcoreMesh(
        core_axis_name="c", subcore_axis_name="s", num_cores=NC, num_subcores=NS
    )

    @functools.partial(
        pl.kernel,
        out_type=jax.ShapeDtypeStruct((NC, n_nodes, d), jnp.float32),
        mesh=mesh,
        scratch_types=[
            pltpu.VMEM((NTRI, SUPER, CHUNK), jnp.int32),
            pltpu.VMEM((NTRI, SUPER, CHUNK), jnp.int32),
            pltpu.VMEM((NTRI, SUPER, CHUNK), jnp.float32),
            pltpu.VMEM((NBUF, CHUNK, d), jnp.float32),
            pltpu.VMEM_SHARED((n_nodes, d), jnp.float32),
            pltpu.SemaphoreType.DMA((NTRI,)),
            pltpu.SemaphoreType.DMA((NBUF,)),
            pltpu.SemaphoreType.DMA((NBUF,)),
            pltpu.SemaphoreType.DMA,
        ],
    )
    def sc_kernel(src_hbm, dst_hbm, w_hbm, ego_hbm, out_hbm,
                  src_v, dst_v, w_v, rows_v, acc, isem, gsem, ssem, zsem):
        cid = lax.axis_index("c")
        sid = lax.axis_index("s")
        wid = sid * NC + cid
        row_start = sid * base_rows
        tail_off = row_start + n_zdma * zrows

        def start_super(t):
            slot = t % NTRI
            sem = isem.at[slot]
            pltpu.async_copy(src_hbm.at[wid, t], src_v.at[slot], sem)
            pltpu.async_copy(dst_hbm.at[wid, t], dst_v.at[slot], sem)
            pltpu.async_copy(w_hbm.at[wid, t], w_v.at[slot], sem)

        def wait_super(t):
            slot = t % NTRI
            sem = isem.at[slot]
            pltpu.make_async_copy(src_hbm.at[wid, t], src_v.at[slot], sem).wait()
            pltpu.make_async_copy(dst_hbm.at[wid, t], dst_v.at[slot], sem).wait()
            pltpu.make_async_copy(w_hbm.at[wid, t], w_v.at[slot], sem).wait()

        def start_gather(c, buf):
            pltpu.async_copy(
                ego_hbm.at[src_v.at[(c // SUPER) % NTRI, c % SUPER]],
                rows_v.at[buf], gsem.at[buf])

        def wait_gather(c, buf):
            pltpu.make_async_copy(
                ego_hbm.at[src_v.at[(c // SUPER) % NTRI, c % SUPER]],
                rows_v.at[buf], gsem.at[buf]).wait()

        start_super(0)
        start_super(1)

        def zero_row(i, _):
            for j in range(d // L):
                rows_v[NBUF - 1, i, pl.ds(j * L, L)] = jnp.zeros((L,), jnp.float32)
            return 0
        lax.fori_loop(0, zrows, zero_row, 0)
        zsrc = rows_v.at[NBUF - 1]
        for q in range(n_zdma):
            pltpu.async_copy(zsrc.at[pl.ds(0, zrows)],
                             acc.at[pl.ds(row_start + q * zrows, zrows)], zsem)

        @pl.when(sid == NS - 1)
        def _():
            pltpu.async_copy(zsrc.at[pl.ds(0, last_rem)],
                             acc.at[pl.ds(tail_off, last_rem)], zsem)

        @pl.when(sid != NS - 1)
        def _():
            pltpu.async_copy(zsrc.at[pl.ds(0, rem)],
                             acc.at[pl.ds(tail_off, rem)], zsem)

        wait_super(0)
        start_gather(0, 0)
        start_gather(1, 1)

        for q in range(n_zdma):
            pltpu.make_async_copy(
                zsrc.at[pl.ds(0, zrows)],
                acc.at[pl.ds(row_start + q * zrows, zrows)], zsem).wait()

        @pl.when(sid == NS - 1)
        def _():
            pltpu.make_async_copy(zsrc.at[pl.ds(0, last_rem)],
                                  acc.at[pl.ds(tail_off, last_rem)], zsem).wait()

        @pl.when(sid != NS - 1)
        def _():
            pltpu.make_async_copy(zsrc.at[pl.ds(0, rem)],
                                  acc.at[pl.ds(tail_off, rem)], zsem).wait()
        plsc.subcore_barrier()

        def super_body(t, _):
            @pl.when(t + 2 < n_super)
            def _():
                start_super(t + 2)

            @pl.when(t + 1 < n_super)
            def _():
                wait_super(t + 1)

            for k in range(SUPER):
                c = t * SUPER + k
                b = c % NBUF
                tri = t % NTRI
                wait_gather(c, b)

                @pl.when(c + 2 < n_chunks)
                def _():
                    nb = (c + 2) % NBUF

                    @pl.when(c >= 1)
                    def _():
                        pltpu.make_async_copy(
                            rows_v.at[nb], acc.at[dst_v.at[tri, k]],
                            ssem.at[nb]).wait()
                    start_gather(c + 2, nb)

                @plsc.parallel_loop(0, CHUNK, step=1, unroll=10)
                def scale_row(r):
                    w16 = w_v[tri, k, pl.ds((r // L) * L, L)]
                    wsplat = w16.at[jnp.broadcast_to(r % L, (L,))].get(
                        mode="promise_in_bounds")
                    for j in range(d // L):
                        sl = pl.ds(j * L, L)
                        rows_v[b, r, sl] = rows_v[b, r, sl] * wsplat

                pltpu.async_copy(
                    rows_v.at[b], acc.at[dst_v.at[tri, k]], ssem.at[b], add=True)
            return 0
        lax.fori_loop(0, n_super, super_body, 0)
        for i in range(3):
            b = (n_chunks - 3 + i) % NBUF
            pltpu.make_async_copy(
                rows_v.at[b], acc.at[dst_v.at[0, 0]], ssem.at[b]).wait()
        plsc.subcore_barrier()

        for q in range(n_zdma):
            sl = pl.ds(row_start + q * zrows, zrows)
            pltpu.async_copy(acc.at[sl], out_hbm.at[cid, sl], zsem)

        @pl.when(sid == NS - 1)
        def _():
            sl = pl.ds(tail_off, last_rem)
            pltpu.async_copy(acc.at[sl], out_hbm.at[cid, sl], zsem)

        @pl.when(sid != NS - 1)
        def _():
            sl = pl.ds(tail_off, rem)
            pltpu.async_copy(acc.at[sl], out_hbm.at[cid, sl], zsem)
        for q in range(n_zdma):
            sl = pl.ds(row_start + q * zrows, zrows)
            pltpu.make_async_copy(acc.at[sl], out_hbm.at[cid, sl], zsem).wait()

        @pl.when(sid == NS - 1)
        def _():
            sl = pl.ds(tail_off, last_rem)
            pltpu.make_async_copy(acc.at[sl], out_hbm.at[cid, sl], zsem).wait()

        @pl.when(sid != NS - 1)
        def _():
            sl = pl.ds(tail_off, rem)
            pltpu.make_async_copy(acc.at[sl], out_hbm.at[cid, sl], zsem).wait()

    return sc_kernel


def _tc_combine(ego, p0, p1, W1, b1, W2, b2):
    n, d = ego.shape
    blk = 400
    assert n % blk == 0

    def body(ego_r, p0_r, p1_r, w1_r, b1_r, w2_r, b2_r, out_r):
        side = p0_r[...] + p1_r[...]
        e = ego_r[...]
        s = jnp.dot(e + side, w1_r[...], preferred_element_type=jnp.float32) + b1_r[...]
        t = jnp.dot(e * side, w2_r[...], preferred_element_type=jnp.float32) + b2_r[...]
        out_r[...] = jnp.where(s >= 0, s, 0.01 * s) + jnp.where(t >= 0, t, 0.01 * t)

    row_spec = pl.BlockSpec((blk, d), lambda i: (i, 0))
    full_spec = pl.BlockSpec((d, d), lambda i: (0, 0))
    vec_spec = pl.BlockSpec((1, d), lambda i: (0, 0))
    return pl.pallas_call(
        body,
        grid=(n // blk,),
        in_specs=[row_spec, row_spec, row_spec, full_spec, vec_spec, full_spec, vec_spec],
        out_specs=row_spec,
        out_shape=jax.ShapeDtypeStruct((n, d), jnp.float32),
    )(ego, p0, p1, W1, b1.reshape(1, d), W2, b2.reshape(1, d))


def kernel(ego_embeddings, edge_index, edge_weight, W1, b1, W2, b2):
    n, d = ego_embeddings.shape
    e = edge_index.shape[1]
    e_per_w = e // NW
    n_super = e_per_w // (CHUNK * SUPER)
    src = edge_index[0].reshape(NW, n_super, SUPER, CHUNK)
    dst = edge_index[1].reshape(NW, n_super, SUPER, CHUNK)
    w = edge_weight.reshape(NW, n_super, SUPER, CHUNK)
    partials = _sc_side_partials(n, e, d)(src, dst, w, ego_embeddings)
    return _tc_combine(ego_embeddings, partials[0], partials[1], W1, b1, W2, b2)


# speedup vs baseline: 1.0093x; 1.0093x over previous
import functools

import jax
import jax.numpy as jnp
from jax import lax
from jax.experimental import pallas as pl
from jax.experimental.pallas import tpu as pltpu
from jax.experimental.pallas import tpu_sc as plsc

NC = 2
NS = 16
L = 16
NW = NC * NS

CHUNK = 80
SUPER = 5
NTRI = 3
NBUF = 3


def _sc_side_partials(n_nodes: int, n_edges: int, d: int):
    assert d % L == 0
    assert n_edges % (NW * CHUNK * SUPER) == 0
    e_per_w = n_edges // NW
    n_chunks = e_per_w // CHUNK
    n_super = n_chunks // SUPER
    assert n_chunks >= 4
    assert n_nodes % 16 == 0
    base_rows = (n_nodes // NS) // 16 * 16
    zrows = CHUNK
    n_zdma = base_rows // zrows
    rem = base_rows - n_zdma * zrows
    last_rem = n_nodes - (NS - 1) * base_rows - n_zdma * zrows
    assert base_rows % 8 == 0 and rem % 8 == 0 and last_rem % 8 == 0
    assert 0 < rem <= zrows and 0 < last_rem <= zrows

    mesh = plsc.VectorSubcoreMesh(
        core_axis_name="c", subcore_axis_name="s", num_cores=NC, num_subcores=NS
    )

    @functools.partial(
        pl.kernel,
        out_type=jax.ShapeDtypeStruct((NC, n_nodes, d), jnp.float32),
        mesh=mesh,
        scratch_types=[
            pltpu.VMEM((NTRI, SUPER, CHUNK), jnp.int32),
            pltpu.VMEM((NTRI, SUPER, CHUNK), jnp.int32),
            pltpu.VMEM((NTRI, SUPER, CHUNK), jnp.float32),
            pltpu.VMEM((NBUF, CHUNK, d), jnp.float32),
            pltpu.VMEM_SHARED((n_nodes, d), jnp.float32),
            pltpu.SemaphoreType.DMA((NTRI,)),
            pltpu.SemaphoreType.DMA((NBUF,)),
            pltpu.SemaphoreType.DMA((NBUF,)),
            pltpu.SemaphoreType.DMA,
        ],
    )
    def sc_kernel(src_hbm, dst_hbm, w_hbm, ego_hbm, out_hbm,
                  src_v, dst_v, w_v, rows_v, acc, isem, gsem, ssem, zsem):
        cid = lax.axis_index("c")
        sid = lax.axis_index("s")
        wid = sid * NC + cid
        row_start = sid * base_rows
        tail_off = row_start + n_zdma * zrows

        def start_super(t):
            slot = t % NTRI
            sem = isem.at[slot]
            pltpu.async_copy(src_hbm.at[wid, t], src_v.at[slot], sem)
            pltpu.async_copy(dst_hbm.at[wid, t], dst_v.at[slot], sem)
            pltpu.async_copy(w_hbm.at[wid, t], w_v.at[slot], sem)

        def wait_super(t):
            slot = t % NTRI
            sem = isem.at[slot]
            pltpu.make_async_copy(src_hbm.at[wid, t], src_v.at[slot], sem).wait()
            pltpu.make_async_copy(dst_hbm.at[wid, t], dst_v.at[slot], sem).wait()
            pltpu.make_async_copy(w_hbm.at[wid, t], w_v.at[slot], sem).wait()

        def start_gather(c, buf):
            pltpu.async_copy(
                ego_hbm.at[src_v.at[(c // SUPER) % NTRI, c % SUPER]],
                rows_v.at[buf], gsem.at[buf])

        def wait_gather(c, buf):
            pltpu.make_async_copy(
                ego_hbm.at[src_v.at[(c // SUPER) % NTRI, c % SUPER]],
                rows_v.at[buf], gsem.at[buf]).wait()

        start_super(0)
        start_super(1)

        def zero_row(i, _):
            for j in range(d // L):
                rows_v[NBUF - 1, i, pl.ds(j * L, L)] = jnp.zeros((L,), jnp.float32)
            return 0
        lax.fori_loop(0, zrows, zero_row, 0)
        zsrc = rows_v.at[NBUF - 1]
        for q in range(n_zdma):
            pltpu.async_copy(zsrc, acc.at[pl.ds(row_start + q * zrows, zrows)], zsem)

        @pl.when(sid == NS - 1)
        def _():
            pltpu.async_copy(zsrc.at[pl.ds(0, last_rem)],
                             acc.at[pl.ds(tail_off, last_rem)], zsem)

        @pl.when(sid != NS - 1)
        def _():
            pltpu.async_copy(zsrc.at[pl.ds(0, rem)],
                             acc.at[pl.ds(tail_off, rem)], zsem)

        wait_super(0)
        start_gather(0, 0)
        start_gather(1, 1)

        for q in range(n_zdma):
            pltpu.make_async_copy(
                zsrc, acc.at[pl.ds(row_start + q * zrows, zrows)], zsem).wait()

        @pl.when(sid == NS - 1)
        def _():
            pltpu.make_async_copy(zsrc.at[pl.ds(0, last_rem)],
                                  acc.at[pl.ds(tail_off, last_rem)], zsem).wait()

        @pl.when(sid != NS - 1)
        def _():
            pltpu.make_async_copy(zsrc.at[pl.ds(0, rem)],
                                  acc.at[pl.ds(tail_off, rem)], zsem).wait()
        plsc.subcore_barrier()

        def super_body(t, _):
            @pl.when(t + 2 < n_super)
            def _():
                start_super(t + 2)

            @pl.when(t + 1 < n_super)
            def _():
                wait_super(t + 1)

            for k in range(SUPER):
                c = t * SUPER + k
                b = c % NBUF
                tri = t % NTRI
                wait_gather(c, b)

                @pl.when(c + 2 < n_chunks)
                def _():
                    nb = (c + 2) % NBUF

                    @pl.when(c >= 1)
                    def _():
                        pltpu.make_async_copy(
                            rows_v.at[nb], acc.at[dst_v.at[tri, k]],
                            ssem.at[nb]).wait()
                    start_gather(c + 2, nb)

                @plsc.parallel_loop(0, CHUNK, step=1, unroll=8)
                def scale_row(r):
                    w16 = w_v[tri, k, pl.ds((r // L) * L, L)]
                    wsplat = w16.at[jnp.broadcast_to(r % L, (L,))].get(
                        mode="promise_in_bounds")
                    for j in range(d // L):
                        sl = pl.ds(j * L, L)
                        rows_v[b, r, sl] = rows_v[b, r, sl] * wsplat

                pltpu.async_copy(
                    rows_v.at[b], acc.at[dst_v.at[tri, k]], ssem.at[b], add=True)
            return 0
        lax.fori_loop(0, n_super, super_body, 0)
        for i in range(3):
            b = (n_chunks - 3 + i) % NBUF
            pltpu.make_async_copy(
                rows_v.at[b], acc.at[dst_v.at[0, 0]], ssem.at[b]).wait()
        plsc.subcore_barrier()

        for q in range(n_zdma):
            sl = pl.ds(row_start + q * zrows, zrows)
            pltpu.async_copy(acc.at[sl], out_hbm.at[cid, sl], zsem)

        @pl.when(sid == NS - 1)
        def _():
            sl = pl.ds(tail_off, last_rem)
            pltpu.async_copy(acc.at[sl], out_hbm.at[cid, sl], zsem)

        @pl.when(sid != NS - 1)
        def _():
            sl = pl.ds(tail_off, rem)
            pltpu.async_copy(acc.at[sl], out_hbm.at[cid, sl], zsem)
        for q in range(n_zdma):
            sl = pl.ds(row_start + q * zrows, zrows)
            pltpu.make_async_copy(acc.at[sl], out_hbm.at[cid, sl], zsem).wait()

        @pl.when(sid == NS - 1)
        def _():
            sl = pl.ds(tail_off, last_rem)
            pltpu.make_async_copy(acc.at[sl], out_hbm.at[cid, sl], zsem).wait()

        @pl.when(sid != NS - 1)
        def _():
            sl = pl.ds(tail_off, rem)
            pltpu.make_async_copy(acc.at[sl], out_hbm.at[cid, sl], zsem).wait()

    return sc_kernel


def _tc_combine(ego, p0, p1, W1, b1, W2, b2):
    n, d = ego.shape
    blk = 400
    assert n % blk == 0

    def body(ego_r, p0_r, p1_r, w1_r, b1_r, w2_r, b2_r, out_r):
        side = p0_r[...] + p1_r[...]
        e = ego_r[...]
        s = jnp.dot(e + side, w1_r[...], preferred_element_type=jnp.float32) + b1_r[...]
        t = jnp.dot(e * side, w2_r[...], preferred_element_type=jnp.float32) + b2_r[...]
        out_r[...] = jnp.where(s >= 0, s, 0.01 * s) + jnp.where(t >= 0, t, 0.01 * t)

    row_spec = pl.BlockSpec((blk, d), lambda i: (i, 0))
    full_spec = pl.BlockSpec((d, d), lambda i: (0, 0))
    vec_spec = pl.BlockSpec((1, d), lambda i: (0, 0))
    return pl.pallas_call(
        body,
        grid=(n // blk,),
        in_specs=[row_spec, row_spec, row_spec, full_spec, vec_spec, full_spec, vec_spec],
        out_specs=row_spec,
        out_shape=jax.ShapeDtypeStruct((n, d), jnp.float32),
    )(ego, p0, p1, W1, b1.reshape(1, d), W2, b2.reshape(1, d))


def kernel(ego_embeddings, edge_index, edge_weight, W1, b1, W2, b2):
    n, d = ego_embeddings.shape
    e = edge_index.shape[1]
    e_per_w = e // NW
    n_super = e_per_w // (CHUNK * SUPER)
    src = edge_index[0].reshape(NW, n_super, SUPER, CHUNK)
    dst = edge_index[1].reshape(NW, n_super, SUPER, CHUNK)
    w = edge_weight.reshape(NW, n_super, SUPER, CHUNK)
    partials = _sc_side_partials(n, e, d)(src, dst, w, ego_embeddings)
    return _tc_combine(ego_embeddings, partials[0], partials[1], W1, b1, W2, b2)
